# baseline (speedup 1.0000x reference)
import jax
import jax.numpy as jnp
from jax.experimental import pallas as pl
from jax.experimental.pallas import tpu as pltpu


def _pre_kernel(x_ref, w1_ref, w2_ref, y1_ref, y2_ref):
    xb = x_ref[...].astype(jnp.bfloat16)
    dn = (((1,), (0,)), ((), ()))
    y1_ref[...] = jax.lax.dot_general(
        xb, w1_ref[...], dn, preferred_element_type=jnp.float32)
    y2 = jax.lax.dot_general(
        xb, w2_ref[...], dn, preferred_element_type=jnp.float32)
    y2_ref[...] = y2.astype(jnp.bfloat16)


def _agg_kernel(adj_ref, y2_ref, y1_ref, out_ref):
    a = adj_ref[...].astype(jnp.bfloat16)
    out_ref[...] = y1_ref[...] + jax.lax.dot_general(
        a, y2_ref[...], (((1,), (0,)), ((), ())),
        preferred_element_type=jnp.float32)


def kernel(input, adj, weight):
    n, f_in = input.shape
    f_out = weight.shape[1]
    w1 = weight[:f_in].astype(jnp.bfloat16)
    w2 = weight[f_in:].astype(jnp.bfloat16)

    bm_pre = min(n, 2000)
    y1, y2 = pl.pallas_call(
        _pre_kernel,
        grid=(n // bm_pre,),
        in_specs=[
            pl.BlockSpec((bm_pre, f_in), lambda i: (i, 0)),
            pl.BlockSpec((f_in, f_out), lambda i: (0, 0)),
            pl.BlockSpec((f_in, f_out), lambda i: (0, 0)),
        ],
        out_specs=[
            pl.BlockSpec((bm_pre, f_out), lambda i: (i, 0)),
            pl.BlockSpec((bm_pre, f_out), lambda i: (i, 0)),
        ],
        out_shape=[
            jax.ShapeDtypeStruct((n, f_out), jnp.float32),
            jax.ShapeDtypeStruct((n, f_out), jnp.bfloat16),
        ],
    )(input, w1, w2)

    bm = min(n, 400)
    out = pl.pallas_call(
        _agg_kernel,
        grid=(n // bm,),
        in_specs=[
            pl.BlockSpec((bm, n), lambda i: (i, 0)),
            pl.BlockSpec((n, f_out), lambda i: (0, 0)),
            pl.BlockSpec((bm, f_out), lambda i: (i, 0)),
        ],
        out_specs=pl.BlockSpec((bm, f_out), lambda i: (i, 0)),
        out_shape=jax.ShapeDtypeStruct((n, f_out), jnp.float32),
        compiler_params=pltpu.CompilerParams(
            dimension_semantics=("arbitrary",)),
    )(adj, y2, y1)
    return out


# speedup vs baseline: 1.1413x; 1.1413x over previous
import jax
import jax.numpy as jnp
from jax.experimental import pallas as pl
from jax.experimental.pallas import tpu as pltpu

_DN = (((1,), (0,)), ((), ()))


def _sage_kernel(adj_ref, x_ref, w_ref, out_ref, y2_ref, *, bm):
    i = pl.program_id(0)
    f_in = x_ref.shape[1]

    @pl.when(i == 0)
    def _build_y2():
        w2 = w_ref[pl.ds(f_in, f_in), :].astype(jnp.bfloat16)
        xb = x_ref[...].astype(jnp.bfloat16)
        y2_ref[...] = jax.lax.dot_general(
            xb, w2, _DN, preferred_element_type=jnp.float32
        ).astype(jnp.bfloat16)

    w1 = w_ref[pl.ds(0, f_in), :].astype(jnp.bfloat16)
    x_band = x_ref[pl.ds(i * bm, bm), :].astype(jnp.bfloat16)
    self_term = jax.lax.dot_general(
        x_band, w1, _DN, preferred_element_type=jnp.float32)
    a = adj_ref[...].astype(jnp.bfloat16)
    out_ref[...] = self_term + jax.lax.dot_general(
        a, y2_ref[...], _DN, preferred_element_type=jnp.float32)


def kernel(input, adj, weight):
    n, f_in = input.shape
    f_out = weight.shape[1]
    bm = min(n, 400)

    import functools
    body = functools.partial(_sage_kernel, bm=bm)
    out = pl.pallas_call(
        body,
        grid=(n // bm,),
        in_specs=[
            pl.BlockSpec((bm, n), lambda i: (i, 0)),
            pl.BlockSpec((n, f_in), lambda i: (0, 0)),
            pl.BlockSpec((2 * f_in, f_out), lambda i: (0, 0)),
        ],
        out_specs=pl.BlockSpec((bm, f_out), lambda i: (i, 0)),
        out_shape=jax.ShapeDtypeStruct((n, f_out), jnp.float32),
        scratch_shapes=[pltpu.VMEM((n, f_out), jnp.bfloat16)],
        compiler_params=pltpu.CompilerParams(
            dimension_semantics=("arbitrary",)),
    )(adj, input, weight)
    return out
